# baseline (speedup 1.0000x reference)
import jax
import jax.numpy as jnp
from jax.experimental import pallas as pl
from jax.experimental.pallas import tpu as pltpu

_B, _T, _D, _K, _NS = 2, 4096, 1024, 7, 3
_BT = 256


def _fused_body(c0, c1, c2, c3, c4, c5, c6, wb, wt, cq, out, gscr):
    i = pl.program_id(0)
    p = jax.lax.rem(i, 2)

    def _select():
        cb = [c[:].astype(jnp.bfloat16) for c in (c0, c1, c2, c3, c4, c5, c6)]
        wbv = wb[:]

        s = [
            jax.lax.dot_general(
                wbv, c, (((1,), (1,)), ((), ())),
                preferred_element_type=jnp.float32,
            )
            for c in cb
        ]

        neg = jnp.float32(-jnp.inf)
        for slot in range(_NS):
            m = s[0]
            am = jnp.zeros(m.shape, jnp.int32)
            for k in range(1, _K):
                gt = s[k] > m
                m = jnp.where(gt, s[k], m)
                am = jnp.where(gt, k, am)
            amc = jnp.transpose(am, (1, 0))
            g = cb[0]
            for k in range(1, _K):
                g = jnp.where(amc == k, cb[k], g)
            gscr[p, slot] = g
            if slot < _NS - 1:
                s = [jnp.where(am == k, neg, s[k]) for k in range(_K)]

    def _project():
        q = jax.lax.rem(i + 1, 2)
        wtb = wt[:]
        for slot in range(_NS):
            g = gscr[q, slot]
            y = jax.lax.dot(g, wtb, preferred_element_type=jnp.float32)
            out[slot] = y + cq[slot]

    _select()
    _project()


def kernel(c0, c1, c2, c3, c4, c5, c6, content_queries, scorer_w, scorer_b, anchor_w):
    del scorer_b
    bt_total = _B * _T
    cands = [c.reshape(bt_total, _D) for c in (c0, c1, c2, c3, c4, c5, c6)]
    wb = scorer_w.astype(jnp.bfloat16)
    wt = anchor_w.T.astype(jnp.bfloat16)
    cq = content_queries.reshape(_NS, 1, _D)

    nblk = bt_total // _BT
    cand_spec = pl.BlockSpec((_BT, _D), lambda i: (jnp.minimum(i, nblk - 1), 0))
    out = pl.pallas_call(
        _fused_body,
        grid=(nblk + 1,),
        in_specs=[cand_spec] * _K
        + [
            pl.BlockSpec((1, _D), lambda i: (0, 0)),
            pl.BlockSpec((_D, _D), lambda i: (0, 0)),
            pl.BlockSpec((_NS, 1, _D), lambda i: (0, 0, 0)),
        ],
        out_specs=pl.BlockSpec(
            (_NS, _BT, _D), lambda i: (0, jnp.maximum(i - 1, 0), 0)
        ),
        out_shape=jax.ShapeDtypeStruct((_NS, bt_total, _D), jnp.float32),
        scratch_shapes=[pltpu.VMEM((2, _NS, _BT, _D), jnp.bfloat16)],
        compiler_params=pltpu.CompilerParams(
            dimension_semantics=("arbitrary",),
        ),
    )(*cands, wb, wt, cq)
    return out.transpose(1, 0, 2)


# speedup vs baseline: 1.0861x; 1.0861x over previous
import jax
import jax.numpy as jnp
from jax.experimental import pallas as pl
from jax.experimental.pallas import tpu as pltpu

_B, _T, _D, _K, _NS = 2, 4096, 1024, 7, 3
_BT = 512


def _fused_body(c0, c1, c2, c3, c4, c5, c6, wb, wt, cq, out):
    cb = [c[:].astype(jnp.bfloat16) for c in (c0, c1, c2, c3, c4, c5, c6)]
    wbv = wb[:]

    s = [
        jnp.transpose(
            jax.lax.dot(c, wbv, preferred_element_type=jnp.float32), (1, 0)
        )
        for c in cb
    ]

    neg = jnp.float32(-jnp.inf)
    slot_idx = []
    for _slot in range(_NS):
        m = s[0]
        am = jnp.zeros(m.shape, jnp.int32)
        for k in range(1, _K):
            gt = s[k] > m
            m = jnp.where(gt, s[k], m)
            am = jnp.where(gt, k, am)
        slot_idx.append(jnp.transpose(am, (1, 0)))
        s = [jnp.where(am == k, neg, s[k]) for k in range(_K)]

    gs = [cb[0]] * _NS
    for k in range(1, _K):
        ck = cb[k]
        gs = [jnp.where(slot_idx[s] == k, ck, gs[s]) for s in range(_NS)]

    wtb = wt[:]
    for slot in range(_NS):
        y = jax.lax.dot(gs[slot], wtb, preferred_element_type=jnp.float32)
        out[slot] = y + cq[slot]


def kernel(c0, c1, c2, c3, c4, c5, c6, content_queries, scorer_w, scorer_b, anchor_w):
    del scorer_b
    bt_total = _B * _T
    cands = [c.reshape(bt_total, _D) for c in (c0, c1, c2, c3, c4, c5, c6)]
    wb = scorer_w.reshape(_D, 1).astype(jnp.bfloat16)
    wt = anchor_w.T.astype(jnp.bfloat16)
    cq = content_queries.reshape(_NS, 1, _D)

    cand_spec = pl.BlockSpec((_BT, _D), lambda i: (i, 0))
    out = pl.pallas_call(
        _fused_body,
        grid=(bt_total // _BT,),
        in_specs=[cand_spec] * _K
        + [
            pl.BlockSpec((_D, 1), lambda i: (0, 0)),
            pl.BlockSpec((_D, _D), lambda i: (0, 0)),
            pl.BlockSpec((_NS, 1, _D), lambda i: (0, 0, 0)),
        ],
        out_specs=pl.BlockSpec((_NS, _BT, _D), lambda i: (0, i, 0)),
        out_shape=jax.ShapeDtypeStruct((_NS, bt_total, _D), jnp.float32),
        compiler_params=pltpu.CompilerParams(
            dimension_semantics=("parallel",),
        ),
    )(*cands, wb, wt, cq)
    return out.transpose(1, 0, 2)


# speedup vs baseline: 1.1232x; 1.0341x over previous
import jax
import jax.numpy as jnp
from jax.experimental import pallas as pl
from jax.experimental.pallas import tpu as pltpu

_B, _T, _D, _K, _NS = 2, 4096, 1024, 7, 3
_BT = 512
_CH = 128


def _fused_body(c0, c1, c2, c3, c4, c5, c6, wb, wt, cq, out):
    wbv = wb[:]
    wtb = wt[:]

    def _prologue(c):
        rows = pl.ds(c * _CH, _CH)
        cb = [
            r[rows, :].astype(jnp.bfloat16)
            for r in (c0, c1, c2, c3, c4, c5, c6)
        ]

        s = [
            jnp.transpose(
                jax.lax.dot(x, wbv, preferred_element_type=jnp.float32),
                (1, 0),
            )
            for x in cb
        ]

        neg = jnp.float32(-jnp.inf)
        slot_idx = []
        for _slot in range(_NS):
            m = s[0]
            am = jnp.zeros(m.shape, jnp.int32)
            for k in range(1, _K):
                gt = s[k] > m
                m = jnp.where(gt, s[k], m)
                am = jnp.where(gt, k, am)
            slot_idx.append(jnp.transpose(am, (1, 0)))
            s = [jnp.where(am == k, neg, s[k]) for k in range(_K)]

        gs = [cb[0]] * _NS
        for k in range(1, _K):
            ck = cb[k]
            gs = [jnp.where(slot_idx[t] == k, ck, gs[t]) for t in range(_NS)]
        return rows, gs

    def _project(rows, gs, slot):
        y = jax.lax.dot(gs[slot], wtb, preferred_element_type=jnp.float32)
        out[slot, rows, :] = y + cq[slot]

    nch = _BT // _CH
    pend = _prologue(0)
    for c in range(1, nch):
        nxt = _prologue(c)
        for slot in range(_NS):
            _project(*pend, slot)
        pend = nxt
    for slot in range(_NS):
        _project(*pend, slot)


def kernel(c0, c1, c2, c3, c4, c5, c6, content_queries, scorer_w, scorer_b, anchor_w):
    del scorer_b
    bt_total = _B * _T
    cands = [c.reshape(bt_total, _D) for c in (c0, c1, c2, c3, c4, c5, c6)]
    wb = scorer_w.reshape(_D, 1).astype(jnp.bfloat16)
    wt = anchor_w.T.astype(jnp.bfloat16)
    cq = content_queries.reshape(_NS, 1, _D)

    cand_spec = pl.BlockSpec((_BT, _D), lambda i: (i, 0))
    out = pl.pallas_call(
        _fused_body,
        grid=(bt_total // _BT,),
        in_specs=[cand_spec] * _K
        + [
            pl.BlockSpec((_D, 1), lambda i: (0, 0)),
            pl.BlockSpec((_D, _D), lambda i: (0, 0)),
            pl.BlockSpec((_NS, 1, _D), lambda i: (0, 0, 0)),
        ],
        out_specs=pl.BlockSpec((_NS, _BT, _D), lambda i: (0, i, 0)),
        out_shape=jax.ShapeDtypeStruct((_NS, bt_total, _D), jnp.float32),
        compiler_params=pltpu.CompilerParams(
            dimension_semantics=("parallel",),
        ),
    )(*cands, wb, wt, cq)
    return out.transpose(1, 0, 2)
